# initial kernel scaffold (unmeasured)
import jax
import jax.numpy as jnp
from jax import lax
from jax.experimental import pallas as pl
from jax.experimental.pallas import tpu as pltpu

N_DEV = 16
N_TOK = 1024
D = 512
H = 1024
E_LOCAL = 4
E_TOT = 64

_MESH = pl.DeviceIdType.MESH


def kernel(x, router_W, route_idx, expert_W):
    def body(x_ref, rw_ref, idx_ref, ew_ref, out_ref,
             xall, wall, ewb, sbuf, rbuf,
             xsend, xrecv, wsend, wrecv, asend, arecv, credit):
        me = lax.axis_index("i")
        left = lax.rem(me - 1 + N_DEV, N_DEV)
        right = lax.rem(me + 1, N_DEV)

        bsem = pltpu.get_barrier_semaphore()
        for nbr in (left, right):
            pl.semaphore_signal(bsem, inc=1, device_id=(nbr,),
                                device_id_type=_MESH)
        pl.semaphore_wait(bsem, 2)

        xf = x_ref[:, :]
        scores = jnp.dot(xf, rw_ref[:, :], preferred_element_type=jnp.float32)
        m = jnp.max(scores, axis=1, keepdims=True)
        p = jnp.exp(scores - m)
        p = p / jnp.sum(p, axis=1, keepdims=True)
        iota = lax.broadcasted_iota(jnp.int32, (N_TOK, E_TOT), 1)
        oh0 = (iota == idx_ref[:, 0:1]).astype(jnp.float32)
        oh1 = (iota == idx_ref[:, 1:2]).astype(jnp.float32)
        g0 = jnp.sum(p * oh0, axis=1, keepdims=True)
        g1 = jnp.sum(p * oh1, axis=1, keepdims=True)
        gs = g0 + g1
        wvec = oh0 * (g0 / gs) + oh1 * (g1 / gs)

        xall[0] = xf.astype(jnp.bfloat16)
        wall[0] = wvec
        ewb[...] = ew_ref[...].astype(jnp.bfloat16)

        ag = []
        for h in range(N_DEV - 1):
            rx = pltpu.make_async_remote_copy(
                src_ref=xall.at[h], dst_ref=xall.at[h + 1],
                send_sem=xsend.at[h], recv_sem=xrecv.at[h],
                device_id=(right,), device_id_type=_MESH)
            rw = pltpu.make_async_remote_copy(
                src_ref=wall.at[h], dst_ref=wall.at[h + 1],
                send_sem=wsend.at[h], recv_sem=wrecv.at[h],
                device_id=(right,), device_id_type=_MESH)
            rx.start()
            rw.start()
            if h > 0:
                ag[h - 1][0].wait_send()
                ag[h - 1][1].wait_send()
            rx.wait_recv()
            rw.wait_recv()
            ag.append((rx, rw))
        ag[-1][0].wait_send()
        ag[-1][1].wait_send()

        descr = [None] * N_DEV
        for s in range(N_DEV):
            slot = (s + 1) % N_DEV
            wc = wall[slot]
            c = jnp.zeros((N_TOK, H), jnp.float32)
            for le in range(E_LOCAL):
                ge = me * E_LOCAL + le
                wle = jnp.sum(wc * (iota == ge).astype(jnp.float32),
                              axis=1, keepdims=True)
                xs = xall[slot] * wle.astype(jnp.bfloat16)
                c = c + jnp.dot(xs, ewb[le],
                                preferred_element_type=jnp.float32)
            if s > 0:
                descr[s - 1].wait_recv()
                c = c + rbuf[(s - 1) % 2].astype(jnp.float32)
                if s <= 13:
                    pl.semaphore_signal(credit.at[(s - 1) % 2], inc=1,
                                        device_id=(left,),
                                        device_id_type=_MESH)
            if s < N_DEV - 1:
                if s >= 2:
                    pl.semaphore_wait(credit.at[s % 2], 1)
                    descr[s - 2].wait_send()
                sbuf[s % 2] = c.astype(jnp.bfloat16)
                r = pltpu.make_async_remote_copy(
                    src_ref=sbuf.at[s % 2], dst_ref=rbuf.at[s % 2],
                    send_sem=asend.at[s], recv_sem=arecv.at[s],
                    device_id=(right,), device_id_type=_MESH)
                r.start()
                descr[s] = r
            else:
                out_ref[:, :] = c
        descr[13].wait_send()
        descr[14].wait_send()

    return pl.pallas_call(
        body,
        out_shape=jax.ShapeDtypeStruct((N_TOK, H), jnp.float32),
        in_specs=[pl.BlockSpec(memory_space=pltpu.VMEM)] * 4,
        out_specs=pl.BlockSpec(memory_space=pltpu.VMEM),
        scratch_shapes=[
            pltpu.VMEM((N_DEV, N_TOK, D), jnp.bfloat16),
            pltpu.VMEM((N_DEV, N_TOK, E_TOT), jnp.float32),
            pltpu.VMEM((E_LOCAL, D, H), jnp.bfloat16),
            pltpu.VMEM((2, N_TOK, H), jnp.bfloat16),
            pltpu.VMEM((2, N_TOK, H), jnp.bfloat16),
            pltpu.SemaphoreType.DMA((N_DEV - 1,)),
            pltpu.SemaphoreType.DMA((N_DEV - 1,)),
            pltpu.SemaphoreType.DMA((N_DEV - 1,)),
            pltpu.SemaphoreType.DMA((N_DEV - 1,)),
            pltpu.SemaphoreType.DMA((N_DEV - 1,)),
            pltpu.SemaphoreType.DMA((N_DEV - 1,)),
            pltpu.SemaphoreType.REGULAR((2,)),
        ],
        compiler_params=pltpu.CompilerParams(collective_id=0),
    )(x, router_W, route_idx, expert_W)


# baseline (device time: 681532 ns/iter reference)
import jax
import jax.numpy as jnp
from jax import lax
from jax.experimental import pallas as pl
from jax.experimental.pallas import tpu as pltpu

N_DEV = 16
N_TOK = 1024
D = 512
H = 1024
E_LOCAL = 4
E_TOT = 64

_MESH = pl.DeviceIdType.MESH


def kernel(x, router_W, route_idx, expert_W):
    def body(x_ref, rw_ref, idx_ref, ew_ref, out_ref,
             xall, wall, ewb, sbuf, rbuf,
             xsend, xrecv, wsend, wrecv, asend, arecv, credit):
        me = lax.axis_index("i")
        left = lax.rem(me - 1 + N_DEV, N_DEV)
        right = lax.rem(me + 1, N_DEV)

        bsem = pltpu.get_barrier_semaphore()
        for nbr in (left, right):
            pl.semaphore_signal(bsem, inc=1, device_id=(nbr,),
                                device_id_type=_MESH)
        pl.semaphore_wait(bsem, 2)

        xf = x_ref[:, :]
        scores = jnp.dot(xf, rw_ref[:, :], preferred_element_type=jnp.float32)
        m = jnp.max(scores, axis=1, keepdims=True)
        p = jnp.exp(scores - m)
        p = p / jnp.sum(p, axis=1, keepdims=True)
        iota = lax.broadcasted_iota(jnp.int32, (N_TOK, E_TOT), 1)
        oh0 = (iota == idx_ref[:, 0:1]).astype(jnp.float32)
        oh1 = (iota == idx_ref[:, 1:2]).astype(jnp.float32)
        g0 = jnp.sum(p * oh0, axis=1, keepdims=True)
        g1 = jnp.sum(p * oh1, axis=1, keepdims=True)
        gs = g0 + g1
        wvec = oh0 * (g0 / gs) + oh1 * (g1 / gs)

        xall[0] = xf.astype(jnp.bfloat16)
        wall[0] = wvec
        ewb[...] = ew_ref[...].astype(jnp.bfloat16)

        ag = []
        for h in range(N_DEV - 1):
            rx = pltpu.make_async_remote_copy(
                src_ref=xall.at[h], dst_ref=xall.at[h + 1],
                send_sem=xsend.at[h], recv_sem=xrecv.at[h],
                device_id=(right,), device_id_type=_MESH)
            rw = pltpu.make_async_remote_copy(
                src_ref=wall.at[h], dst_ref=wall.at[h + 1],
                send_sem=wsend.at[h], recv_sem=wrecv.at[h],
                device_id=(right,), device_id_type=_MESH)
            rx.start()
            rw.start()
            if h > 0:
                ag[h - 1][0].wait_send()
                ag[h - 1][1].wait_send()
            rx.wait_recv()
            rw.wait_recv()
            ag.append((rx, rw))
        ag[-1][0].wait_send()
        ag[-1][1].wait_send()

        descr = [None] * N_DEV
        for s in range(N_DEV):
            slot = (s + 1) % N_DEV
            wc = wall[slot]
            c = jnp.zeros((N_TOK, H), jnp.float32)
            for le in range(E_LOCAL):
                ge = me * E_LOCAL + le
                wle = jnp.sum(wc * (iota == ge).astype(jnp.float32),
                              axis=1, keepdims=True)
                xs = xall[slot] * wle.astype(jnp.bfloat16)
                c = c + jnp.dot(xs, ewb[le],
                                preferred_element_type=jnp.float32)
            if s > 0:
                descr[s - 1].wait_recv()
                c = c + rbuf[(s - 1) % 2].astype(jnp.float32)
                if s <= 13:
                    pl.semaphore_signal(credit.at[(s - 1) % 2], inc=1,
                                        device_id=(left,),
                                        device_id_type=_MESH)
            if s < N_DEV - 1:
                if s >= 2:
                    pl.semaphore_wait(credit.at[s % 2], 1)
                    descr[s - 2].wait_send()
                sbuf[s % 2] = c.astype(jnp.bfloat16)
                r = pltpu.make_async_remote_copy(
                    src_ref=sbuf.at[s % 2], dst_ref=rbuf.at[s % 2],
                    send_sem=asend.at[s], recv_sem=arecv.at[s],
                    device_id=(right,), device_id_type=_MESH)
                r.start()
                descr[s] = r
            else:
                out_ref[:, :] = c
        descr[13].wait_send()
        descr[14].wait_send()

    return pl.pallas_call(
        body,
        out_shape=jax.ShapeDtypeStruct((N_TOK, H), jnp.float32),
        in_specs=[pl.BlockSpec(memory_space=pltpu.VMEM)] * 4,
        out_specs=pl.BlockSpec(memory_space=pltpu.VMEM),
        scratch_shapes=[
            pltpu.VMEM((N_DEV, N_TOK, D), jnp.bfloat16),
            pltpu.VMEM((N_DEV, N_TOK, E_TOT), jnp.float32),
            pltpu.VMEM((E_LOCAL, D, H), jnp.bfloat16),
            pltpu.VMEM((2, N_TOK, H), jnp.bfloat16),
            pltpu.VMEM((2, N_TOK, H), jnp.bfloat16),
            pltpu.SemaphoreType.DMA((N_DEV - 1,)),
            pltpu.SemaphoreType.DMA((N_DEV - 1,)),
            pltpu.SemaphoreType.DMA((N_DEV - 1,)),
            pltpu.SemaphoreType.DMA((N_DEV - 1,)),
            pltpu.SemaphoreType.DMA((N_DEV - 1,)),
            pltpu.SemaphoreType.DMA((N_DEV - 1,)),
            pltpu.SemaphoreType.REGULAR((2,)),
        ],
        compiler_params=pltpu.CompilerParams(
            collective_id=0, vmem_limit_bytes=100 * 1024 * 1024),
    )(x, router_W, route_idx, expert_W)


# device time: 315783 ns/iter; 2.1582x vs baseline; 2.1582x over previous
import jax
import jax.numpy as jnp
from jax import lax
from jax.experimental import pallas as pl
from jax.experimental.pallas import tpu as pltpu

N_DEV = 16
N_TOK = 1024
HALF = N_TOK // 2
D = 512
H = 1024
E_LOCAL = 4
E_TOT = 64
XW = D + E_TOT

_MESH = pl.DeviceIdType.MESH


def kernel(x, router_W, route_idx, expert_W):
    def body(x_ref, rw_ref, idx_ref, ew_ref, out_ref,
             xwR, xwL, ewb,
             sbufR, rbufR, sbufL, rbufL,
             agsendR, agrecvR, agsendL, agrecvL,
             asendR, arecvR, asendL, arecvL,
             creditR, creditL):
        me = lax.axis_index("i")
        left = lax.rem(me - 1 + N_DEV, N_DEV)
        right = lax.rem(me + 1, N_DEV)

        bsem = pltpu.get_barrier_semaphore()
        for nbr in (left, right):
            pl.semaphore_signal(bsem, inc=1, device_id=(nbr,),
                                device_id_type=_MESH)
        pl.semaphore_wait(bsem, 2)

        xf = x_ref[:, :]
        scores = jnp.dot(xf, rw_ref[:, :], preferred_element_type=jnp.float32)
        m = jnp.max(scores, axis=1, keepdims=True)
        p = jnp.exp(scores - m)
        p = p / jnp.sum(p, axis=1, keepdims=True)
        iota = lax.broadcasted_iota(jnp.int32, (N_TOK, E_TOT), 1)
        oh0 = (iota == idx_ref[:, 0:1]).astype(jnp.float32)
        oh1 = (iota == idx_ref[:, 1:2]).astype(jnp.float32)
        g0 = jnp.sum(p * oh0, axis=1, keepdims=True)
        g1 = jnp.sum(p * oh1, axis=1, keepdims=True)
        gs = g0 + g1
        wvec = oh0 * (g0 / gs) + oh1 * (g1 / gs)

        packed = jnp.concatenate(
            [xf, wvec], axis=1).astype(jnp.bfloat16)
        xwR[0] = packed[0:HALF, :]
        xwL[0] = packed[HALF:N_TOK, :]
        ewb[...] = jnp.reshape(
            ew_ref[...].astype(jnp.bfloat16), (E_LOCAL * D, H))

        iota_h = lax.broadcasted_iota(jnp.int32, (HALF, E_TOT), 1)

        agR = []
        agL = []

        def start_ag(h):
            rR = pltpu.make_async_remote_copy(
                src_ref=xwR.at[h], dst_ref=xwR.at[h + 1],
                send_sem=agsendR.at[h], recv_sem=agrecvR.at[h],
                device_id=(right,), device_id_type=_MESH)
            rL = pltpu.make_async_remote_copy(
                src_ref=xwL.at[h], dst_ref=xwL.at[h + 1],
                send_sem=agsendL.at[h], recv_sem=agrecvL.at[h],
                device_id=(left,), device_id_type=_MESH)
            rR.start()
            rL.start()
            agR.append(rR)
            agL.append(rL)

        start_ag(0)

        def contribution(xw_slot):
            xc = xw_slot[:, 0:D]
            wc = xw_slot[:, D:XW].astype(jnp.float32)
            parts = []
            for le in range(E_LOCAL):
                ge = me * E_LOCAL + le
                wle = jnp.sum(wc * (iota_h == ge).astype(jnp.float32),
                              axis=1, keepdims=True)
                parts.append(xc * wle.astype(jnp.bfloat16))
            xbig = jnp.concatenate(parts, axis=1)
            return jnp.dot(xbig, ewb[...],
                           preferred_element_type=jnp.float32)

        dR = [None] * N_DEV
        dL = [None] * N_DEV
        for s in range(N_DEV):
            slot = (s + 1) % N_DEV
            if s < N_DEV - 1:
                agR[s].wait_recv()
                agL[s].wait_recv()
                if s < N_DEV - 2:
                    agR[s].wait_send()
                    agL[s].wait_send()
                    start_ag(s + 1)
            cR = contribution(xwR[slot])
            cL = contribution(xwL[slot])
            if s > 0:
                dR[s - 1].wait_recv()
                cR = cR + rbufR[(s - 1) % 2].astype(jnp.float32)
                dL[s - 1].wait_recv()
                cL = cL + rbufL[(s - 1) % 2].astype(jnp.float32)
                if s <= 13:
                    pl.semaphore_signal(creditR.at[(s - 1) % 2], inc=1,
                                        device_id=(left,),
                                        device_id_type=_MESH)
                    pl.semaphore_signal(creditL.at[(s - 1) % 2], inc=1,
                                        device_id=(right,),
                                        device_id_type=_MESH)
            if s < N_DEV - 1:
                if s >= 2:
                    pl.semaphore_wait(creditR.at[s % 2], 1)
                    dR[s - 2].wait_send()
                    pl.semaphore_wait(creditL.at[s % 2], 1)
                    dL[s - 2].wait_send()
                sbufR[s % 2] = cR.astype(jnp.bfloat16)
                sbufL[s % 2] = cL.astype(jnp.bfloat16)
                rR = pltpu.make_async_remote_copy(
                    src_ref=sbufR.at[s % 2], dst_ref=rbufR.at[s % 2],
                    send_sem=asendR.at[s], recv_sem=arecvR.at[s],
                    device_id=(right,), device_id_type=_MESH)
                rL = pltpu.make_async_remote_copy(
                    src_ref=sbufL.at[s % 2], dst_ref=rbufL.at[s % 2],
                    send_sem=asendL.at[s], recv_sem=arecvL.at[s],
                    device_id=(left,), device_id_type=_MESH)
                rR.start()
                rL.start()
                dR[s] = rR
                dL[s] = rL
            else:
                out_ref[0:HALF, :] = cR
                out_ref[HALF:N_TOK, :] = cL
        for d in (dR[13], dR[14], dL[13], dL[14], agR[14], agL[14]):
            d.wait_send()

    nhop = N_DEV - 1
    return pl.pallas_call(
        body,
        out_shape=jax.ShapeDtypeStruct((N_TOK, H), jnp.float32),
        in_specs=[pl.BlockSpec(memory_space=pltpu.VMEM)] * 4,
        out_specs=pl.BlockSpec(memory_space=pltpu.VMEM),
        scratch_shapes=[
            pltpu.VMEM((N_DEV, HALF, XW), jnp.bfloat16),
            pltpu.VMEM((N_DEV, HALF, XW), jnp.bfloat16),
            pltpu.VMEM((E_LOCAL * D, H), jnp.bfloat16),
            pltpu.VMEM((2, HALF, H), jnp.bfloat16),
            pltpu.VMEM((2, HALF, H), jnp.bfloat16),
            pltpu.VMEM((2, HALF, H), jnp.bfloat16),
            pltpu.VMEM((2, HALF, H), jnp.bfloat16),
            pltpu.SemaphoreType.DMA((nhop,)),
            pltpu.SemaphoreType.DMA((nhop,)),
            pltpu.SemaphoreType.DMA((nhop,)),
            pltpu.SemaphoreType.DMA((nhop,)),
            pltpu.SemaphoreType.DMA((nhop,)),
            pltpu.SemaphoreType.DMA((nhop,)),
            pltpu.SemaphoreType.DMA((nhop,)),
            pltpu.SemaphoreType.DMA((nhop,)),
            pltpu.SemaphoreType.REGULAR((2,)),
            pltpu.SemaphoreType.REGULAR((2,)),
        ],
        compiler_params=pltpu.CompilerParams(
            collective_id=0, vmem_limit_bytes=100 * 1024 * 1024),
    )(x, router_W, route_idx, expert_W)


# device time: 315304 ns/iter; 2.1615x vs baseline; 1.0015x over previous
import jax
import jax.numpy as jnp
from jax import lax
from jax.experimental import pallas as pl
from jax.experimental.pallas import tpu as pltpu

N_DEV = 16
N_TOK = 1024
HALF = N_TOK // 2
D = 512
H = 1024
E_LOCAL = 4
E_TOT = 64
XW = D + E_TOT

_MESH = pl.DeviceIdType.MESH


def kernel(x, router_W, route_idx, expert_W):
    def body(x_ref, rw_ref, idx_ref, ew_ref, out_ref,
             xwR, xwL, ewb,
             sbufR, rbufR, sbufL, rbufL,
             agsendR, agrecvR, agsendL, agrecvL,
             asendR, arecvR, asendL, arecvL,
             creditR, creditL):
        me = lax.axis_index("i")
        left = lax.rem(me - 1 + N_DEV, N_DEV)
        right = lax.rem(me + 1, N_DEV)

        bsem = pltpu.get_barrier_semaphore()
        for nbr in (left, right):
            pl.semaphore_signal(bsem, inc=1, device_id=(nbr,),
                                device_id_type=_MESH)
        pl.semaphore_wait(bsem, 2)

        xf = x_ref[:, :]
        scores = jnp.dot(xf, rw_ref[:, :], preferred_element_type=jnp.float32)
        m = jnp.max(scores, axis=1, keepdims=True)
        p = jnp.exp(scores - m)
        p = p / jnp.sum(p, axis=1, keepdims=True)
        iota = lax.broadcasted_iota(jnp.int32, (N_TOK, E_TOT), 1)
        oh0 = (iota == idx_ref[:, 0:1]).astype(jnp.float32)
        oh1 = (iota == idx_ref[:, 1:2]).astype(jnp.float32)
        g0 = jnp.sum(p * oh0, axis=1, keepdims=True)
        g1 = jnp.sum(p * oh1, axis=1, keepdims=True)
        gs = g0 + g1
        wvec = oh0 * (g0 / gs) + oh1 * (g1 / gs)

        packed = jnp.concatenate(
            [xf, wvec], axis=1).astype(jnp.bfloat16)
        xwR[0] = packed[0:HALF, :]
        xwL[0] = packed[HALF:N_TOK, :]

        iota_h = lax.broadcasted_iota(jnp.int32, (HALF, E_TOT), 1)

        agR = []
        agL = []

        def start_ag(h):
            rR = pltpu.make_async_remote_copy(
                src_ref=xwR.at[h], dst_ref=xwR.at[h + 1],
                send_sem=agsendR.at[h], recv_sem=agrecvR.at[h],
                device_id=(right,), device_id_type=_MESH)
            rL = pltpu.make_async_remote_copy(
                src_ref=xwL.at[h], dst_ref=xwL.at[h + 1],
                send_sem=agsendL.at[h], recv_sem=agrecvL.at[h],
                device_id=(left,), device_id_type=_MESH)
            rR.start()
            rL.start()
            agR.append(rR)
            agL.append(rL)

        start_ag(0)
        ewb[...] = jnp.reshape(
            ew_ref[...].astype(jnp.bfloat16), (E_LOCAL * D, H))

        def contribution(xw_slot):
            xc = xw_slot[:, 0:D]
            wc = xw_slot[:, D:XW].astype(jnp.float32)
            parts = []
            for le in range(E_LOCAL):
                ge = me * E_LOCAL + le
                wle = jnp.sum(wc * (iota_h == ge).astype(jnp.float32),
                              axis=1, keepdims=True)
                parts.append(xc * wle.astype(jnp.bfloat16))
            xbig = jnp.concatenate(parts, axis=1)
            return jnp.dot(xbig, ewb[...],
                           preferred_element_type=jnp.float32)

        agR[0].wait_recv()
        agL[0].wait_recv()
        agR[0].wait_send()
        agL[0].wait_send()
        start_ag(1)
        cR = contribution(xwR[1])
        cL = contribution(xwL[1])

        dR = [None] * N_DEV
        dL = [None] * N_DEV
        for s in range(N_DEV):
            if s > 0:
                dR[s - 1].wait_recv()
                cR = cR + rbufR[(s - 1) % 2].astype(jnp.float32)
                dL[s - 1].wait_recv()
                cL = cL + rbufL[(s - 1) % 2].astype(jnp.float32)
                if s <= 13:
                    pl.semaphore_signal(creditR.at[(s - 1) % 2], inc=1,
                                        device_id=(left,),
                                        device_id_type=_MESH)
                    pl.semaphore_signal(creditL.at[(s - 1) % 2], inc=1,
                                        device_id=(right,),
                                        device_id_type=_MESH)
            if s < N_DEV - 1:
                if s >= 2:
                    pl.semaphore_wait(creditR.at[s % 2], 1)
                    dR[s - 2].wait_send()
                    pl.semaphore_wait(creditL.at[s % 2], 1)
                    dL[s - 2].wait_send()
                sbufR[s % 2] = cR.astype(jnp.bfloat16)
                sbufL[s % 2] = cL.astype(jnp.bfloat16)
                rR = pltpu.make_async_remote_copy(
                    src_ref=sbufR.at[s % 2], dst_ref=rbufR.at[s % 2],
                    send_sem=asendR.at[s], recv_sem=arecvR.at[s],
                    device_id=(right,), device_id_type=_MESH)
                rL = pltpu.make_async_remote_copy(
                    src_ref=sbufL.at[s % 2], dst_ref=rbufL.at[s % 2],
                    send_sem=asendL.at[s], recv_sem=arecvL.at[s],
                    device_id=(left,), device_id_type=_MESH)
                rR.start()
                rL.start()
                dR[s] = rR
                dL[s] = rL
            else:
                out_ref[0:HALF, :] = cR
                out_ref[HALF:N_TOK, :] = cL
            if s < N_DEV - 1:
                if s <= 13:
                    agR[s + 1].wait_recv()
                    agL[s + 1].wait_recv()
                slot_next = (s + 2) % N_DEV
                cR = contribution(xwR[slot_next])
                cL = contribution(xwL[slot_next])
                if s <= 12:
                    if s >= 1:
                        agR[s].wait_send()
                        agL[s].wait_send()
                    start_ag(s + 2)
        for d in (dR[13], dR[14], dL[13], dL[14],
                  agR[13], agL[13], agR[14], agL[14]):
            d.wait_send()

    nhop = N_DEV - 1
    return pl.pallas_call(
        body,
        out_shape=jax.ShapeDtypeStruct((N_TOK, H), jnp.float32),
        in_specs=[pl.BlockSpec(memory_space=pltpu.VMEM)] * 4,
        out_specs=pl.BlockSpec(memory_space=pltpu.VMEM),
        scratch_shapes=[
            pltpu.VMEM((N_DEV, HALF, XW), jnp.bfloat16),
            pltpu.VMEM((N_DEV, HALF, XW), jnp.bfloat16),
            pltpu.VMEM((E_LOCAL * D, H), jnp.bfloat16),
            pltpu.VMEM((2, HALF, H), jnp.bfloat16),
            pltpu.VMEM((2, HALF, H), jnp.bfloat16),
            pltpu.VMEM((2, HALF, H), jnp.bfloat16),
            pltpu.VMEM((2, HALF, H), jnp.bfloat16),
            pltpu.SemaphoreType.DMA((nhop,)),
            pltpu.SemaphoreType.DMA((nhop,)),
            pltpu.SemaphoreType.DMA((nhop,)),
            pltpu.SemaphoreType.DMA((nhop,)),
            pltpu.SemaphoreType.DMA((nhop,)),
            pltpu.SemaphoreType.DMA((nhop,)),
            pltpu.SemaphoreType.DMA((nhop,)),
            pltpu.SemaphoreType.DMA((nhop,)),
            pltpu.SemaphoreType.REGULAR((2,)),
            pltpu.SemaphoreType.REGULAR((2,)),
        ],
        compiler_params=pltpu.CompilerParams(
            collective_id=0, vmem_limit_bytes=100 * 1024 * 1024),
    )(x, router_W, route_idx, expert_W)
